# baseline (speedup 1.0000x reference)
import functools

import jax
import jax.numpy as jnp
import numpy as np
from jax import lax
from jax.experimental import pallas as pl
from jax.experimental.pallas import tpu as pltpu
from jax.experimental.pallas import tpu_sc as plsc

_N = 10000
_E = 160000
_HID = 256
_H2 = 128
_NG = 50
_NAT = 100
_CUT = 5.0
_NL = 4

_step = np.float32(_CUT) * (np.float32(1.0) / np.float32(_NG - 1))
_COEFF = np.float32(-0.5 / float(_step) ** 2)

_BN = 1000
_BE = 2000


def _init_body(x_ref, fc_ref, emb_ref, Wc_ref, bc_ref, Wct_ref, Wcb_ref,
               bcomb_ref, h0_ref, h1_ref):
    xv = x_ref[...]
    ids = lax.broadcasted_iota(jnp.int32, (_BN, _NAT + 1), 1)
    onehot = (xv == ids).astype(jnp.float32)
    h_atom = jnp.dot(onehot, emb_ref[...], preferred_element_type=jnp.float32)
    fc = fc_ref[...]
    Wc = Wc_ref[...]
    h_coord = (fc[:, 0:1] * Wc[0:1, :] + fc[:, 1:2] * Wc[1:2, :]
               + fc[:, 2:3] * Wc[2:3, :] + bc_ref[...])
    h = (jnp.dot(h_atom, Wct_ref[...], preferred_element_type=jnp.float32)
         + jnp.dot(h_coord, Wcb_ref[...], preferred_element_type=jnp.float32)
         + bcomb_ref[...])
    h0_ref[...] = h[:, :_H2]
    h1_ref[...] = h[:, _H2:]


def _init_call(x2, fc, emb, Wc, bc, Wct, Wcb, bcomb):
    full = lambda a: pl.BlockSpec(a.shape, lambda i: (0,) * a.ndim)
    return pl.pallas_call(
        _init_body,
        grid=(_N // _BN,),
        in_specs=[
            pl.BlockSpec((_BN, 1), lambda i: (i, 0)),
            pl.BlockSpec((_BN, 3), lambda i: (i, 0)),
            full(emb), full(Wc), full(bc), full(Wct), full(Wcb), full(bcomb),
        ],
        out_specs=[
            pl.BlockSpec((_BN, _H2), lambda i: (i, 0)),
            pl.BlockSpec((_BN, _H2), lambda i: (i, 0)),
        ],
        out_shape=[
            jax.ShapeDtypeStruct((_N, _H2), jnp.float32),
            jax.ShapeDtypeStruct((_N, _H2), jnp.float32),
        ],
    )(x2, fc, emb, Wc, bc, Wct, Wcb, bcomb)


def _edge_body(ev_ref, off_ref, eW1_ref, eb1_ref, eW2_ref, eb2_ref, *out_refs):
    ev = ev_ref[...]
    d2 = ev[:, 0:1] * ev[:, 0:1] + ev[:, 1:2] * ev[:, 1:2] + ev[:, 2:3] * ev[:, 2:3]
    dist = jnp.sqrt(d2)
    diff = dist - off_ref[...]
    feat = jnp.exp(_COEFF * diff * diff)
    for l in range(_NL):
        t = jnp.dot(feat, eW1_ref[l], preferred_element_type=jnp.float32) + eb1_ref[l]
        a = t * (1.0 / (1.0 + jnp.exp(-t)))
        w = jnp.dot(a, eW2_ref[l], preferred_element_type=jnp.float32) + eb2_ref[l]
        out_refs[2 * l][...] = w[:, :_H2]
        out_refs[2 * l + 1][...] = w[:, _H2:]


def _edge_call(ev, off, eW1s, eb1s, eW2s, eb2s):
    full = lambda a: pl.BlockSpec(a.shape, lambda i: (0,) * a.ndim)
    wspec = pl.BlockSpec((_BE, _H2), lambda i: (i, 0))
    wshape = jax.ShapeDtypeStruct((_E, _H2), jnp.float32)
    return pl.pallas_call(
        _edge_body,
        grid=(_E // _BE,),
        in_specs=[
            pl.BlockSpec((_BE, 3), lambda i: (i, 0)),
            full(off), full(eW1s), full(eb1s), full(eW2s), full(eb2s),
        ],
        out_specs=[wspec] * (2 * _NL),
        out_shape=[wshape] * (2 * _NL),
    )(ev, off, eW1s, eb1s, eW2s, eb2s)


_NSUB = 16
_ES = _E // _NSUB
_K = 80
_NCHUNK = _ES // _K
_NP = 10240
_NROWS = _NP // _NSUB


@functools.cache
def _build_msg_kernel():
    return functools.partial(
        pl.kernel,
        mesh=plsc.VectorSubcoreMesh(core_axis_name="c", subcore_axis_name="s"),
        out_type=(
            jax.ShapeDtypeStruct((_NP, _H2), jnp.float32),
            jax.ShapeDtypeStruct((_NP, _H2), jnp.float32),
        ),
        scratch_types=(
            pltpu.VMEM((2, _K), jnp.int32),
            pltpu.VMEM((2, _K), jnp.int32),
            pltpu.VMEM((2, _K, _H2), jnp.float32),
            pltpu.VMEM((2, _K, _H2), jnp.float32),
            pltpu.VMEM_SHARED((_NP, _H2), jnp.float32),
            pltpu.SemaphoreType.DMA,
            pltpu.SemaphoreType.DMA,
            pltpu.SemaphoreType.DMA,
        ),
    )(_msg_body)


def _msg_body(h0, h1, w0, w1, src, dst, z, agg0, agg1,
              idx_s, idx_d, hbuf, wbuf, acc, isem, gsem, wsem):
    cid = lax.axis_index("c")
    sid = lax.axis_index("s")
    rows = pl.ds(sid * _NROWS, _NROWS)
    pltpu.sync_copy(z.at[rows], acc.at[rows])
    plsc.subcore_barrier()

    def run(h_hbm, w_hbm, agg_hbm):
        def fire_idx(k, slot):
            base = sid * _ES + k * _K
            pltpu.async_copy(src.at[pl.ds(base, _K)], idx_s.at[slot], isem)
            pltpu.async_copy(dst.at[pl.ds(base, _K)], idx_d.at[slot], isem)

        def wait_idx(k, slot):
            base = sid * _ES + k * _K
            pltpu.make_async_copy(src.at[pl.ds(base, _K)], idx_s.at[slot], isem).wait()
            pltpu.make_async_copy(dst.at[pl.ds(base, _K)], idx_d.at[slot], isem).wait()

        def fire_data(k, slot):
            base = sid * _ES + k * _K
            pltpu.async_copy(h_hbm.at[idx_s.at[slot]], hbuf.at[slot], gsem)
            pltpu.async_copy(w_hbm.at[pl.ds(base, _K)], wbuf.at[slot], wsem)

        fire_idx(0, 0)
        wait_idx(0, 0)
        fire_data(0, 0)
        fire_idx(1, 1)

        def chunk(k, carry):
            slot = lax.rem(k, 2)

            @pl.when(k + 1 < _NCHUNK)
            def _():
                wait_idx(k + 1, 1 - slot)
                fire_data(k + 1, 1 - slot)

            base = sid * _ES + k * _K
            pltpu.make_async_copy(h_hbm.at[idx_s.at[slot]], hbuf.at[slot], gsem).wait()
            pltpu.make_async_copy(w_hbm.at[pl.ds(base, _K)], wbuf.at[slot], wsem).wait()

            def row(r, c2):
                for j in range(_H2 // 16):
                    sl = pl.ds(j * 16, 16)
                    hbuf[slot, r, sl] = hbuf[slot, r, sl] * wbuf[slot, r, sl]
                return c2

            lax.fori_loop(0, _K, row, 0)
            pltpu.sync_copy(hbuf.at[slot], acc.at[idx_d.at[slot]], add=True)

            @pl.when(k + 2 < _NCHUNK)
            def _():
                fire_idx(k + 2, slot)

            return carry

        lax.fori_loop(0, _NCHUNK, chunk, 0)
        plsc.subcore_barrier()
        pltpu.sync_copy(acc.at[rows], agg_hbm.at[rows])

    @pl.when(cid == 0)
    def _():
        run(h0, w0, agg0)

    @pl.when(cid == 1)
    def _():
        run(h1, w1, agg1)


def _messages(h0, h1, w0, w1, src, dst, z):
    return _build_msg_kernel()(h0, h1, w0, w1, src, dst, z)


def _node_body(h0_ref, h1_ref, a0_ref, a1_ref, A_ref, B_ref, C_ref, D_ref,
               nb1_ref, nW2_ref, nb2_ref, g_ref, be_ref,
               h_ref, h0n_ref, h1n_ref):
    h0 = h0_ref[...]
    h1 = h1_ref[...]
    u = (jnp.dot(h0, A_ref[...], preferred_element_type=jnp.float32)
         + jnp.dot(h1, B_ref[...], preferred_element_type=jnp.float32)
         + jnp.dot(a0_ref[...], C_ref[...], preferred_element_type=jnp.float32)
         + jnp.dot(a1_ref[...], D_ref[...], preferred_element_type=jnp.float32)
         + nb1_ref[...])
    t = u * (1.0 / (1.0 + jnp.exp(-u)))
    v = jnp.dot(t, nW2_ref[...], preferred_element_type=jnp.float32) + nb2_ref[...]
    r = jnp.concatenate([h0, h1], axis=1) + v
    mu = jnp.mean(r, axis=1, keepdims=True)
    d = r - mu
    var = jnp.mean(d * d, axis=1, keepdims=True)
    hn = d * lax.rsqrt(var + 1e-5) * g_ref[...] + be_ref[...]
    h_ref[...] = hn
    h0n_ref[...] = hn[:, :_H2]
    h1n_ref[...] = hn[:, _H2:]


def _node_call(h0, h1, a0, a1, A, B, C, D, nb1, nW2, nb2, g, be):
    full = lambda a: pl.BlockSpec(a.shape, lambda i: (0,) * a.ndim)
    nspec = pl.BlockSpec((_BN, _H2), lambda i: (i, 0))
    return pl.pallas_call(
        _node_body,
        grid=(_N // _BN,),
        in_specs=[nspec, nspec, nspec, nspec,
                  full(A), full(B), full(C), full(D),
                  full(nb1), full(nW2), full(nb2), full(g), full(be)],
        out_specs=[pl.BlockSpec((_BN, _HID), lambda i: (i, 0)), nspec, nspec],
        out_shape=[
            jax.ShapeDtypeStruct((_N, _HID), jnp.float32),
            jax.ShapeDtypeStruct((_N, _H2), jnp.float32),
            jax.ShapeDtypeStruct((_N, _H2), jnp.float32),
        ],
    )(h0, h1, a0, a1, A, B, C, D, nb1, nW2, nb2, g, be)


def kernel(x, frac_coords, edge_index, edge_vec, batch, params):
    p = params
    x2 = x.reshape(_N, 1).astype(jnp.int32)
    offsets = jnp.linspace(0.0, _CUT, _NG).astype(jnp.float32).reshape(1, _NG)
    bc = p["bc"].reshape(1, _HID)
    bcomb = p["bcomb"].reshape(1, _HID)
    Wct = p["Wcomb"][:_HID]
    Wcb = p["Wcomb"][_HID:]

    h0, h1 = _init_call(x2, frac_coords, p["emb"], p["Wc"], bc, Wct, Wcb, bcomb)

    eW1s = jnp.stack([lp["eW1"] for lp in p["layers"]])
    eb1s = jnp.stack([lp["eb1"].reshape(1, _HID) for lp in p["layers"]])
    eW2s = jnp.stack([lp["eW2"] for lp in p["layers"]])
    eb2s = jnp.stack([lp["eb2"].reshape(1, _HID) for lp in p["layers"]])
    ws = _edge_call(edge_vec, offsets, eW1s, eb1s, eW2s, eb2s)

    src = edge_index[0]
    dst = edge_index[1]
    z = jnp.zeros((_NP, _H2), jnp.float32)

    h = None
    for l, lp in enumerate(p["layers"]):
        a0, a1 = _messages(h0, h1, ws[2 * l], ws[2 * l + 1], src, dst, z)
        nW1 = lp["nW1"]
        h, h0, h1 = _node_call(
            h0, h1, a0, a1,
            nW1[:_H2], nW1[_H2:_HID], nW1[_HID:_HID + _H2], nW1[_HID + _H2:],
            lp["nb1"].reshape(1, _HID), lp["nW2"], lp["nb2"].reshape(1, _HID),
            lp["g"].reshape(1, _HID), lp["be"].reshape(1, _HID))
    return h


# speedup vs baseline: 2.4663x; 2.4663x over previous
import functools

import jax
import jax.numpy as jnp
import numpy as np
from jax import lax
from jax.experimental import pallas as pl
from jax.experimental.pallas import tpu as pltpu
from jax.experimental.pallas import tpu_sc as plsc

_N = 10000
_E = 160000
_HID = 256
_H2 = 128
_NG = 50
_NAT = 100
_CUT = 5.0
_NL = 4

_step = np.float32(_CUT) * (np.float32(1.0) / np.float32(_NG - 1))
_COEFF = np.float32(-0.5 / float(_step) ** 2)

_BN = 1000
_BE = 2000

def _init_body(x_ref, fc_ref, emb_ref, Wc_ref, bc_ref, Wct_ref, Wcb_ref,
               bcomb_ref, h_ref, h2_ref):
    xv = x_ref[...]
    ids = lax.broadcasted_iota(jnp.int32, (_BN, _NAT + 1), 1)
    onehot = (xv == ids).astype(jnp.float32)
    h_atom = jnp.dot(onehot, emb_ref[...], preferred_element_type=jnp.float32)
    fc = fc_ref[...]
    Wc = Wc_ref[...]
    h_coord = (fc[:, 0:1] * Wc[0:1, :] + fc[:, 1:2] * Wc[1:2, :]
               + fc[:, 2:3] * Wc[2:3, :] + bc_ref[...])
    h = (jnp.dot(h_atom, Wct_ref[...], preferred_element_type=jnp.float32)
         + jnp.dot(h_coord, Wcb_ref[...], preferred_element_type=jnp.float32)
         + bcomb_ref[...])
    h_ref[...] = h
    h2_ref[0] = h[:, :_H2]
    h2_ref[1] = h[:, _H2:]


def _init_call(x2, fc, emb, Wc, bc, Wct, Wcb, bcomb):
    full = lambda a: pl.BlockSpec(a.shape, lambda i: (0,) * a.ndim)
    return pl.pallas_call(
        _init_body,
        grid=(_N // _BN,),
        in_specs=[
            pl.BlockSpec((_BN, 1), lambda i: (i, 0)),
            pl.BlockSpec((_BN, 3), lambda i: (i, 0)),
            full(emb), full(Wc), full(bc), full(Wct), full(Wcb), full(bcomb),
        ],
        out_specs=[
            pl.BlockSpec((_BN, _HID), lambda i: (i, 0)),
            pl.BlockSpec((2, _BN, _H2), lambda i: (0, i, 0)),
        ],
        out_shape=[
            jax.ShapeDtypeStruct((_N, _HID), jnp.float32),
            jax.ShapeDtypeStruct((2, _N, _H2), jnp.float32),
        ],
    )(x2, fc, emb, Wc, bc, Wct, Wcb, bcomb)


def _edge_body(ev_ref, off_ref, eW1_ref, eb1_ref, eW2_ref, eb2_ref, w2_ref):
    ev = ev_ref[...]
    d2 = ev[:, 0:1] * ev[:, 0:1] + ev[:, 1:2] * ev[:, 1:2] + ev[:, 2:3] * ev[:, 2:3]
    dist = jnp.sqrt(d2)
    diff = dist - off_ref[...]
    feat = jnp.exp(_COEFF * diff * diff).astype(jnp.bfloat16)
    t = jnp.dot(feat, eW1_ref[...], preferred_element_type=jnp.float32) + eb1_ref[...]
    a = (t * (1.0 / (1.0 + jnp.exp(-t)))).astype(jnp.bfloat16)
    w = jnp.dot(a, eW2_ref[...], preferred_element_type=jnp.float32) + eb2_ref[...]
    w2_ref[0] = w[:, :_H2]
    w2_ref[1] = w[:, _H2:]


def _edge_call(ev, off, eW1, eb1, eW2, eb2):
    full = lambda a: pl.BlockSpec(a.shape, lambda i: (0,) * a.ndim)
    return pl.pallas_call(
        _edge_body,
        grid=(_E // _BE,),
        in_specs=[
            pl.BlockSpec((_BE, 3), lambda i: (i, 0)),
            full(off), full(eW1), full(eb1), full(eW2), full(eb2),
        ],
        out_specs=pl.BlockSpec((2, _BE, _H2), lambda i: (0, i, 0)),
        out_shape=jax.ShapeDtypeStruct((2, _E, _H2), jnp.float32),
    )(ev, off, eW1, eb1, eW2, eb2)


_NSUB = 16
_ES = _E // _NSUB
_K = 40
_NCHUNK = _ES // _K
_NP = 10240
_NROWS = _NP // _NSUB


@functools.cache
def _build_msg_kernel():
    return functools.partial(
        pl.kernel,
        mesh=plsc.VectorSubcoreMesh(core_axis_name="c", subcore_axis_name="s"),
        out_type=jax.ShapeDtypeStruct((2, _NP, _H2), jnp.float32),
        scratch_types=(
            pltpu.VMEM((4, _K), jnp.int32),
            pltpu.VMEM((4, _K), jnp.int32),
            pltpu.VMEM((4, _K), jnp.int32),
            pltpu.VMEM((4, _K, _H2), jnp.float32),
            pltpu.VMEM((4, _K, _H2), jnp.float32),
            pltpu.VMEM_SHARED((_NP, _H2), jnp.float32),
            pltpu.SemaphoreType.DMA,
            pltpu.SemaphoreType.DMA,
            pltpu.SemaphoreType.DMA,
            pltpu.SemaphoreType.DMA,
        ),
    )(_msg_body)


def _msg_body(h2, w2, src, dst, z, agg2,
              idx_s, idx_d, sidx, hbuf, wbuf, acc, isem, gsem, wsem, ssem):
    cid = lax.axis_index("c")
    sid = lax.axis_index("s")
    rows = pl.ds(sid * _NROWS, _NROWS)
    pltpu.sync_copy(z.at[rows], acc.at[rows])
    plsc.subcore_barrier()

    h_hbm = h2.at[cid]
    w_hbm = w2.at[cid]
    agg_hbm = agg2.at[cid]

    def fire_idx(k, slot):
        base = sid * _ES + k * _K
        pltpu.async_copy(src.at[pl.ds(base, _K)], idx_s.at[slot], isem)
        pltpu.async_copy(dst.at[pl.ds(base, _K)], idx_d.at[slot], isem)

    def wait_idx(k, slot):
        base = sid * _ES + k * _K
        pltpu.make_async_copy(src.at[pl.ds(base, _K)], idx_s.at[slot], isem).wait()
        pltpu.make_async_copy(dst.at[pl.ds(base, _K)], idx_d.at[slot], isem).wait()

    def fire_data(k, slot):
        base = sid * _ES + k * _K
        pltpu.async_copy(h_hbm.at[idx_s.at[slot]], hbuf.at[slot], gsem)
        pltpu.async_copy(w_hbm.at[pl.ds(base, _K)], wbuf.at[slot], wsem)

    def drain_scatter():
        pltpu.make_async_copy(hbuf.at[0], acc.at[sidx.at[0]], ssem).wait()

    def process(k, slot):
        @pl.when(k + 2 < _NCHUNK)
        def _():
            wait_idx(k + 2, (slot + 2) % 4)

        @pl.when(k >= 2)
        def _():
            drain_scatter()

        @pl.when(k + 2 < _NCHUNK)
        def _():
            fire_data(k + 2, (slot + 2) % 4)

        base = sid * _ES + k * _K
        pltpu.make_async_copy(h_hbm.at[idx_s.at[slot]], hbuf.at[slot], gsem).wait()
        pltpu.make_async_copy(w_hbm.at[pl.ds(base, _K)], wbuf.at[slot], wsem).wait()

        for a in (0, 16, _K - 16):
            sidx[slot, pl.ds(a, 16)] = idx_d[slot, pl.ds(a, 16)]

        @pl.when(k + 4 < _NCHUNK)
        def _():
            fire_idx(k + 4, slot)

        def rowfn(r, c2):
            for j in range(_H2 // 16):
                sl = pl.ds(j * 16, 16)
                hbuf[slot, r, sl] = hbuf[slot, r, sl] * wbuf[slot, r, sl]
            return c2

        lax.fori_loop(0, _K, rowfn, 0)
        pltpu.async_copy(hbuf.at[slot], acc.at[sidx.at[slot]], ssem, add=True)

    fire_idx(0, 0)
    fire_idx(1, 1)
    wait_idx(0, 0)
    fire_data(0, 0)
    fire_idx(2, 2)
    wait_idx(1, 1)
    fire_data(1, 1)
    fire_idx(3, 3)

    def quad(g, carry):
        for j in range(4):
            process(4 * g + j, j)
        return carry

    lax.fori_loop(0, _NCHUNK // 4, quad, 0)
    for j in range(_NCHUNK % 4):
        kk = _NCHUNK - _NCHUNK % 4 + j
        process(jnp.int32(kk), kk % 4)
    drain_scatter()
    drain_scatter()
    plsc.subcore_barrier()
    pltpu.sync_copy(acc.at[rows], agg_hbm.at[rows])


def _messages(h2, w2, src, dst, z):
    return _build_msg_kernel()(h2, w2, src, dst, z)


def _node_body(h_in_ref, a2_ref, AB_ref, C_ref, D_ref,
               nb1_ref, nW2_ref, nb2_ref, g_ref, be_ref,
               h_ref, h2_ref):
    h = h_in_ref[...]
    u = (jnp.dot(h, AB_ref[...], preferred_element_type=jnp.float32)
         + jnp.dot(a2_ref[0], C_ref[...], preferred_element_type=jnp.float32)
         + jnp.dot(a2_ref[1], D_ref[...], preferred_element_type=jnp.float32)
         + nb1_ref[...])
    t = u * (1.0 / (1.0 + jnp.exp(-u)))
    v = jnp.dot(t, nW2_ref[...], preferred_element_type=jnp.float32) + nb2_ref[...]
    r = h + v
    mu = jnp.mean(r, axis=1, keepdims=True)
    d = r - mu
    var = jnp.mean(d * d, axis=1, keepdims=True)
    hn = d * lax.rsqrt(var + 1e-5) * g_ref[...] + be_ref[...]
    h_ref[...] = hn
    h2_ref[0] = hn[:, :_H2]
    h2_ref[1] = hn[:, _H2:]


def _node_call(h, a2, AB, C, D, nb1, nW2, nb2, g, be):
    full = lambda a: pl.BlockSpec(a.shape, lambda i: (0,) * a.ndim)
    hspec = pl.BlockSpec((_BN, _HID), lambda i: (i, 0))
    return pl.pallas_call(
        _node_body,
        grid=(_N // _BN,),
        in_specs=[hspec,
                  pl.BlockSpec((2, _BN, _H2), lambda i: (0, i, 0)),
                  full(AB), full(C), full(D),
                  full(nb1), full(nW2), full(nb2), full(g), full(be)],
        out_specs=[hspec, pl.BlockSpec((2, _BN, _H2), lambda i: (0, i, 0))],
        out_shape=[
            jax.ShapeDtypeStruct((_N, _HID), jnp.float32),
            jax.ShapeDtypeStruct((2, _N, _H2), jnp.float32),
        ],
    )(h, a2, AB, C, D, nb1, nW2, nb2, g, be)


def kernel(x, frac_coords, edge_index, edge_vec, batch, params):
    p = params
    x2 = x.reshape(_N, 1).astype(jnp.int32)
    offsets = jnp.linspace(0.0, _CUT, _NG).astype(jnp.float32).reshape(1, _NG)
    bc = p["bc"].reshape(1, _HID)
    bcomb = p["bcomb"].reshape(1, _HID)
    Wct = p["Wcomb"][:_HID]
    Wcb = p["Wcomb"][_HID:]

    h, h2 = _init_call(x2, frac_coords, p["emb"], p["Wc"], bc, Wct, Wcb, bcomb)

    src = edge_index[0]
    dst = edge_index[1]
    z = jnp.zeros((_NP, _H2), jnp.float32)

    def edge(l):
        lp = p["layers"][l]
        return _edge_call(edge_vec, offsets,
                          lp["eW1"].astype(jnp.bfloat16),
                          lp["eb1"].reshape(1, _HID),
                          lp["eW2"].astype(jnp.bfloat16),
                          lp["eb2"].reshape(1, _HID))

    w2 = edge(0)
    for l, lp in enumerate(p["layers"]):
        a2 = _messages(h2, w2, src, dst, z)
        if l + 1 < _NL:
            w2 = edge(l + 1)
        nW1 = lp["nW1"]
        h, h2 = _node_call(
            h, a2,
            nW1[:_HID], nW1[_HID:_HID + _H2], nW1[_HID + _H2:],
            lp["nb1"].reshape(1, _HID), lp["nW2"], lp["nb2"].reshape(1, _HID),
            lp["g"].reshape(1, _HID), lp["be"].reshape(1, _HID))
    return h
